# Initial kernel scaffold; baseline (speedup 1.0000x reference)
import jax
import jax.numpy as jnp
from jax.experimental import pallas as pl


def kernel(embeddings_table, length):
    raise NotImplementedError("write your pallas kernel here")


# speedup vs baseline: 12.8096x; 12.8096x over previous
import functools

import jax
import jax.numpy as jnp
from jax import lax
from jax.experimental import pallas as pl
from jax.experimental.pallas import tpu as pltpu
from jax.experimental.pallas import tpu_sc as plsc

D = 128
V = 257
L = 1024
P_ROWS = 2 * L - 1
FILL = L - 129
NC = 2
NS = 16
ROWS_PER_TILE = L // (NC * NS)
FB = 128


def _sc_body(table_hbm, out_hbm, p_sh, fill_v, trow_v):
    c = lax.axis_index("c")
    s = lax.axis_index("s")

    @pl.when(s == 0)
    def _():
        pltpu.sync_copy(table_hbm, p_sh.at[pl.ds(FILL, V)])

    def _build_fill(edge_row):
        pltpu.sync_copy(table_hbm.at[pl.ds(edge_row, 1)], trow_v)

        def rep(r, carry):
            for k in range(D // 16):
                fill_v[r, pl.ds(k * 16, 16)] = trow_v[0, pl.ds(k * 16, 16)]
            return carry

        lax.fori_loop(0, FB, rep, 0)

    @pl.when(s == 1)
    def _():
        _build_fill(0)
        pltpu.sync_copy(fill_v.at[pl.ds(0, FILL % FB)],
                        p_sh.at[pl.ds(0, FILL % FB)])
        for b in range(FILL // FB):
            pltpu.sync_copy(fill_v, p_sh.at[pl.ds(FILL % FB + b * FB, FB)])

    @pl.when(s == 2)
    def _():
        _build_fill(V - 1)
        for b in range(FILL // FB):
            pltpu.sync_copy(fill_v, p_sh.at[pl.ds(FILL + V + b * FB, FB)])
        pltpu.sync_copy(fill_v.at[pl.ds(0, FILL % FB)],
                        p_sh.at[pl.ds(P_ROWS - FILL % FB, FILL % FB)])

    plsc.subcore_barrier()

    base = c * (NS * ROWS_PER_TILE) + s * ROWS_PER_TILE

    def copy_row(k, carry):
        i = base + k
        pltpu.sync_copy(p_sh.at[pl.ds(L - 1 - i, L)], out_hbm.at[i])
        return carry

    lax.fori_loop(0, ROWS_PER_TILE, copy_row, 0)


@functools.partial(
    pl.kernel,
    out_type=jax.ShapeDtypeStruct((L, L, D), jnp.float32),
    mesh=plsc.VectorSubcoreMesh(core_axis_name="c", subcore_axis_name="s"),
    scratch_types=[
        pltpu.VMEM_SHARED((P_ROWS, D), jnp.float32),
        pltpu.VMEM((FB, D), jnp.float32),
        pltpu.VMEM((1, D), jnp.float32),
    ],
)
def _rel_pos_sc(table_hbm, out_hbm, p_sh, fill_v, trow_v):
    _sc_body(table_hbm, out_hbm, p_sh, fill_v, trow_v)


def kernel(embeddings_table, length):
    return _rel_pos_sc(embeddings_table)
